# baseline (speedup 1.0000x reference)
import functools

import jax
import jax.numpy as jnp
from jax import lax
from jax.experimental import pallas as pl
from jax.experimental.pallas import tpu as pltpu
from jax.experimental.pallas import tpu_sc as plsc

N = 10000
E = 320000
D = 128
C = 128
G = 64
FCW = 256

NC = 2
NS = 16
NW = NC * NS
NPAD = 10112
RPT = NPAD // NS
ECH = 64
NCH = 160
EPT = NCH * ECH
EPAD = NW * EPT
NBUF = 4
SDEP = 2
NOUT = NCH // NBUF
LN16 = ECH // 16

K0 = 236
K1 = 2 * NCH - K0
MAXK = K0

_mesh = plsc.VectorSubcoreMesh(core_axis_name="c", subcore_axis_name="s")


def _unpack_idx(combo_v, row, colbase, st_v, b, part):
    for j in range(LN16):
        v = combo_v[row, pl.ds(colbase + j * 16, 16)]
        if part == 0:
            st_v[b, pl.ds(j * 16, 16)] = lax.bitwise_and(v, 0xFFFF)
        else:
            st_v[b, pl.ds(j * 16, 16)] = lax.shift_right_logical(v, 16)


def _sc_deg_body(combo_hbm, ones_hbm, zeros_hbm, out_hbm,
                 combo_v, dst_st, ones_v, acc, sem):
    cid = lax.axis_index("c")
    sid = lax.axis_index("s")
    wid = sid * NC + cid
    pltpu.sync_copy(zeros_hbm, acc.at[pl.ds(sid * RPT, RPT)])
    pltpu.sync_copy(ones_hbm, ones_v)
    pltpu.sync_copy(combo_hbm.at[wid], combo_v)
    plsc.subcore_barrier()

    def round_(o, carry):
        for b in range(NBUF):
            _unpack_idx(combo_v, o * 2 + b // 2, (b % 2) * ECH, dst_st, b, 1)
            pltpu.async_copy(ones_v, acc.at[dst_st.at[b]], sem, add=True)
        for b in range(NBUF):
            pltpu.make_async_copy(ones_hbm, ones_v, sem).wait()
        return carry

    lax.fori_loop(0, NOUT, round_, 0)
    plsc.subcore_barrier()
    pltpu.sync_copy(acc.at[pl.ds(sid * RPT, RPT)],
                    out_hbm.at[cid, pl.ds(sid * RPT, RPT)])


_sc_deg = pl.kernel(
    _sc_deg_body,
    out_type=jax.ShapeDtypeStruct((NC, NPAD, C), jnp.float32),
    mesh=_mesh,
    scratch_types=[
        pltpu.VMEM((NCH // 2, 2 * ECH), jnp.int32),
        pltpu.VMEM((NBUF, ECH), jnp.int32),
        pltpu.VMEM((ECH, C), jnp.float32),
        pltpu.VMEM_SHARED((NPAD, C), jnp.float32),
        pltpu.SemaphoreType.DMA,
    ],
)


def _sc_agg_body(combo_hbm, hs_hbm, zeros_hbm, out_hbm,
                 combo_v, src_st, dst_st, rows_v, acc,
                 gs0, gs1, gs2, gs3, ss0, ss1, ss2, ss3):
    cid = lax.axis_index("c")
    sid = lax.axis_index("s")
    wid = sid * NC + cid
    gsems = (gs0, gs1, gs2, gs3)
    ssems = (ss0, ss1, ss2, ss3)

    kch = jnp.where(cid == 0, K0, K1)
    pltpu.sync_copy(zeros_hbm, acc.at[pl.ds(sid * RPT, RPT)])
    pltpu.sync_copy(combo_hbm.at[wid], combo_v)
    plsc.subcore_barrier()

    def prep_and_gather(b, row, colbase):
        _unpack_idx(combo_v, row, colbase, src_st, b, 0)
        _unpack_idx(combo_v, row, colbase, dst_st, b, 1)
        pltpu.async_copy(hs_hbm.at[src_st.at[b]], rows_v.at[b], gsems[b])

    def gwait(b):
        pltpu.make_async_copy(hs_hbm.at[pl.ds(0, ECH)], rows_v.at[b],
                              gsems[b]).wait()

    def fire_scatter(b):
        pltpu.async_copy(rows_v.at[b], acc.at[dst_st.at[b]], ssems[b], add=True)

    def swait(b):
        pltpu.make_async_copy(hs_hbm.at[pl.ds(0, ECH)], rows_v.at[b],
                              ssems[b]).wait()

    for b in range(NBUF):
        prep_and_gather(b, b // 2, (b % 2) * ECH)

    def round_(o, carry):
        for b in range(NBUF):
            i = o * NBUF + b
            gwait(b)
            fire_scatter(b)
            br = (b - SDEP) % NBUF

            @pl.when(i >= SDEP)
            def _():
                swait(br)

            @pl.when(jnp.logical_and(i >= SDEP, i <= kch - 1 - (NBUF - SDEP)))
            def _():
                prep_and_gather(br, o * 2 + (b + SDEP) // 2, (b % 2) * ECH)
        return carry

    lax.fori_loop(0, kch // NBUF, round_, 0)
    for k in range(SDEP):
        swait((NBUF - SDEP + k) % NBUF)
    plsc.subcore_barrier()
    pltpu.sync_copy(acc.at[pl.ds(sid * RPT, RPT)],
                    out_hbm.at[cid, pl.ds(sid * RPT, RPT)])


_sc_agg = pl.kernel(
    _sc_agg_body,
    out_type=jax.ShapeDtypeStruct((NC, NPAD, C), jnp.float32),
    mesh=_mesh,
    scratch_types=[
        pltpu.VMEM((MAXK // 2, 2 * ECH), jnp.int32),
        pltpu.VMEM((NBUF, ECH), jnp.int32),
        pltpu.VMEM((NBUF, ECH), jnp.int32),
        pltpu.VMEM((NBUF, ECH, C), jnp.float32),
        pltpu.VMEM_SHARED((NPAD, C), jnp.float32),
        pltpu.SemaphoreType.DMA,
        pltpu.SemaphoreType.DMA,
        pltpu.SemaphoreType.DMA,
        pltpu.SemaphoreType.DMA,
        pltpu.SemaphoreType.DMA,
        pltpu.SemaphoreType.DMA,
        pltpu.SemaphoreType.DMA,
        pltpu.SemaphoreType.DMA,
    ],
)


BN = 2528


def _dinv_of(deg_ref):
    deg = deg_ref[0, :, 0:1] + deg_ref[1, :, 0:1] + 1.0
    return lax.rsqrt(deg)


def _tc1_body(x_ref, w_ref, deg_ref, hs_ref):
    h = jnp.dot(x_ref[...], w_ref[...], preferred_element_type=jnp.float32)
    hs_ref[...] = h * _dinv_of(deg_ref)


def _layernorm(t, g, b):
    mu = jnp.mean(t, axis=-1, keepdims=True)
    var = jnp.mean((t - mu) * (t - mu), axis=-1, keepdims=True)
    return (t - mu) * lax.rsqrt(var + 1e-5) * g + b


def _tc2_body(agg_ref, hs_ref, deg_ref, w1_ref, b0_ref, g0_ref, be0_ref,
              x1_ref, hs1_ref):
    dinv = _dinv_of(deg_ref)
    agg = agg_ref[0] + agg_ref[1]
    t = dinv * (agg + hs_ref[...]) + b0_ref[...]
    xb = jnp.maximum(_layernorm(t, g0_ref[...], be0_ref[...]), 0.0)
    x1_ref[...] = xb
    hs1_ref[...] = jnp.dot(xb, w1_ref[...],
                           preferred_element_type=jnp.float32) * dinv


BN3 = 2000


def _tc3_body(agg_ref, hs_ref, deg_ref, x1_ref, batch_ref,
              b1_ref, g1_ref, be1_ref, sums_ref, counts_ref):
    i = pl.program_id(0)

    @pl.when(i == 0)
    def _():
        sums_ref[...] = jnp.zeros_like(sums_ref)
        counts_ref[...] = jnp.zeros_like(counts_ref)

    dinv = _dinv_of(deg_ref)
    agg = agg_ref[0] + agg_ref[1]
    t = dinv * (agg + hs_ref[...]) + b1_ref[...]
    h2 = jnp.maximum(_layernorm(t, g1_ref[...], be1_ref[...]), 0.0) + x1_ref[...]

    iota = lax.broadcasted_iota(jnp.int32, (1, G), 1)
    oh = (batch_ref[...] == iota).astype(jnp.float32)
    dn = (((0,), (0,)), ((), ()))
    sums_ref[...] += lax.dot_general(oh, h2, dn,
                                     preferred_element_type=jnp.float32)
    counts_ref[...] += lax.dot_general(oh, jnp.ones((BN3, C), jnp.float32), dn,
                                       preferred_element_type=jnp.float32)


def _tc4_body(sums_ref, counts_ref, fw1_ref, fb1_ref, fw2_ref, fb2_ref,
              fw3_ref, fb3_ref, out_ref):
    pooled = sums_ref[...] / jnp.maximum(counts_ref[...], 1.0)
    z = jnp.maximum(jnp.dot(pooled, fw1_ref[...],
                            preferred_element_type=jnp.float32) + fb1_ref[...], 0.0)
    z = jnp.maximum(jnp.dot(z, fw2_ref[...],
                            preferred_element_type=jnp.float32) + fb2_ref[...], 0.0)
    z = jnp.dot(z, fw3_ref[...], preferred_element_type=jnp.float32) + fb3_ref[...]
    out_ref[...] = jnp.tanh(z)


def _row_spec(bn, ncols):
    return pl.BlockSpec((bn, ncols), lambda i: (i, 0))


def _half_spec(bn, ncols):
    return pl.BlockSpec((2, bn, ncols), lambda i: (0, i, 0))


def _full_spec(shape):
    return pl.BlockSpec(shape, lambda i: tuple(0 for _ in shape))


_tc1 = pl.pallas_call(
    _tc1_body,
    grid=(NPAD // BN,),
    in_specs=[_row_spec(BN, D), _full_spec((D, D)), _half_spec(BN, C)],
    out_specs=_row_spec(BN, D),
    out_shape=jax.ShapeDtypeStruct((NPAD, D), jnp.float32),
)

_tc2 = pl.pallas_call(
    _tc2_body,
    grid=(NPAD // BN,),
    in_specs=[_half_spec(BN, C), _row_spec(BN, C), _half_spec(BN, C),
              _full_spec((D, C)), _full_spec((1, D)), _full_spec((1, D)),
              _full_spec((1, D))],
    out_specs=[_row_spec(BN, D), _row_spec(BN, C)],
    out_shape=[jax.ShapeDtypeStruct((NPAD, D), jnp.float32),
               jax.ShapeDtypeStruct((NPAD, C), jnp.float32)],
)

_tc3 = pl.pallas_call(
    _tc3_body,
    grid=(N // BN3,),
    in_specs=[_half_spec(BN3, C), _row_spec(BN3, C), _half_spec(BN3, C),
              _row_spec(BN3, C), _row_spec(BN3, 1),
              _full_spec((1, C)), _full_spec((1, C)), _full_spec((1, C))],
    out_specs=[_full_spec((G, C)), _full_spec((G, C))],
    out_shape=[jax.ShapeDtypeStruct((G, C), jnp.float32),
               jax.ShapeDtypeStruct((G, C), jnp.float32)],
)

_tc4 = pl.pallas_call(
    _tc4_body,
    grid=(1,),
    in_specs=[_full_spec((G, C)), _full_spec((G, C)),
              _full_spec((C, FCW)), _full_spec((1, FCW)),
              _full_spec((FCW, FCW)), _full_spec((1, FCW)),
              _full_spec((FCW, C)), _full_spec((1, C))],
    out_specs=_full_spec((G, C)),
    out_shape=jax.ShapeDtypeStruct((G, C), jnp.float32),
)


@jax.jit
def kernel(x, edge_index, batch, W0, b0, g0, be0, W1, b1, g1, be1,
           fW1, fb1, fW2, fb2, fW3, fb3):
    src = edge_index[0].astype(jnp.int32)
    dst = edge_index[1].astype(jnp.int32)
    pad = jnp.full((EPAD - E,), N, jnp.int32)
    src_p = jnp.concatenate([src, pad])
    dst_p = jnp.concatenate([dst, pad])
    flat = src_p | (dst_p << 16)
    combo = flat.reshape(NW, NCH // 2, 2 * ECH)
    n0 = NS * K0 * ECH
    c0 = flat[:n0].reshape(NS, K0 * ECH)
    c1 = flat[n0:].reshape(NS, K1 * ECH)
    trash = jnp.int32(N | (N << 16))
    slab = jnp.full((NW, MAXK * ECH), trash, jnp.int32)
    slab = slab.at[0::2, :K0 * ECH].set(c0)
    slab = slab.at[1::2, :K1 * ECH].set(c1)
    combo_a = slab.reshape(NW, MAXK // 2, 2 * ECH)
    x_p = jnp.pad(x, ((0, NPAD - N), (0, 0)))

    zeros_r = jnp.zeros((RPT, C), jnp.float32)
    ones_c = jnp.ones((ECH, C), jnp.float32)

    deg2 = _sc_deg(combo, ones_c, zeros_r)

    hs0 = _tc1(x_p, W0, deg2)
    agg0 = _sc_agg(combo_a, hs0, zeros_r)
    x1, hs1 = _tc2(agg0, hs0, deg2, W1,
                   b0.reshape(1, D), g0.reshape(1, D), be0.reshape(1, D))
    agg1 = _sc_agg(combo_a, hs1, zeros_r)
    sums, counts = _tc3(agg1, hs1, deg2, x1, batch.reshape(N, 1).astype(jnp.int32),
                        b1.reshape(1, C), g1.reshape(1, C), be1.reshape(1, C))
    return _tc4(sums, counts, fW1, fb1.reshape(1, FCW), fW2, fb2.reshape(1, FCW),
                fW3, fb3.reshape(1, C))


# speedup vs baseline: 1.0135x; 1.0135x over previous
import functools

import jax
import jax.numpy as jnp
from jax import lax
from jax.experimental import pallas as pl
from jax.experimental.pallas import tpu as pltpu
from jax.experimental.pallas import tpu_sc as plsc

N = 10000
E = 320000
D = 128
C = 128
G = 64
FCW = 256

NC = 2
NS = 16
NW = NC * NS
NPAD = 10112
RPT = NPAD // NS
ECH = 64
NCH = 160
EPT = NCH * ECH
EPAD = NW * EPT
NBUF = 4
SDEP = 2
NOUT = NCH // NBUF
LN16 = ECH // 16

K0 = 160
K1 = 2 * NCH - K0
MAXK = K0

_mesh = plsc.VectorSubcoreMesh(core_axis_name="c", subcore_axis_name="s")


def _unpack_idx(combo_v, row, colbase, st_v, b, part):
    for j in range(LN16):
        v = combo_v[row, pl.ds(colbase + j * 16, 16)]
        if part == 0:
            st_v[b, pl.ds(j * 16, 16)] = lax.bitwise_and(v, 0xFFFF)
        else:
            st_v[b, pl.ds(j * 16, 16)] = lax.shift_right_logical(v, 16)


def _sc_deg_body(combo_hbm, ones_hbm, zeros_hbm, out_hbm,
                 combo_v, dst_st, ones_v, acc, sem):
    cid = lax.axis_index("c")
    sid = lax.axis_index("s")
    wid = sid * NC + cid
    pltpu.sync_copy(zeros_hbm, acc.at[pl.ds(sid * RPT, RPT)])
    pltpu.sync_copy(ones_hbm, ones_v)
    pltpu.sync_copy(combo_hbm.at[wid], combo_v)
    plsc.subcore_barrier()

    def round_(o, carry):
        for b in range(NBUF):
            _unpack_idx(combo_v, o * 2 + b // 2, (b % 2) * ECH, dst_st, b, 1)
            pltpu.async_copy(ones_v, acc.at[dst_st.at[b]], sem, add=True)
        for b in range(NBUF):
            pltpu.make_async_copy(ones_hbm, ones_v, sem).wait()
        return carry

    lax.fori_loop(0, NOUT, round_, 0)
    plsc.subcore_barrier()
    pltpu.sync_copy(acc.at[pl.ds(sid * RPT, RPT)],
                    out_hbm.at[cid, pl.ds(sid * RPT, RPT)])


_sc_deg = pl.kernel(
    _sc_deg_body,
    out_type=jax.ShapeDtypeStruct((NC, NPAD, C), jnp.float32),
    mesh=_mesh,
    scratch_types=[
        pltpu.VMEM((NCH // 2, 2 * ECH), jnp.int32),
        pltpu.VMEM((NBUF, ECH), jnp.int32),
        pltpu.VMEM((ECH, C), jnp.float32),
        pltpu.VMEM_SHARED((NPAD, C), jnp.float32),
        pltpu.SemaphoreType.DMA,
    ],
)


def _sc_agg_body(combo_hbm, hs_hbm, zeros_hbm, out_hbm,
                 combo_v, src_st, dst_st, rows_v, acc,
                 gs0, gs1, gs2, gs3, ss0, ss1, ss2, ss3):
    cid = lax.axis_index("c")
    sid = lax.axis_index("s")
    wid = sid * NC + cid
    gsems = (gs0, gs1, gs2, gs3)
    ssems = (ss0, ss1, ss2, ss3)

    kch = jnp.where(cid == 0, K0, K1)
    pltpu.sync_copy(zeros_hbm, acc.at[pl.ds(sid * RPT, RPT)])
    pltpu.sync_copy(combo_hbm.at[wid], combo_v)
    plsc.subcore_barrier()

    def prep_and_gather(b, row, colbase):
        _unpack_idx(combo_v, row, colbase, src_st, b, 0)
        _unpack_idx(combo_v, row, colbase, dst_st, b, 1)
        pltpu.async_copy(hs_hbm.at[src_st.at[b]], rows_v.at[b], gsems[b])

    def gwait(b):
        pltpu.make_async_copy(hs_hbm.at[pl.ds(0, ECH)], rows_v.at[b],
                              gsems[b]).wait()

    def fire_scatter(b):
        pltpu.async_copy(rows_v.at[b], acc.at[dst_st.at[b]], ssems[b], add=True)

    def swait(b):
        pltpu.make_async_copy(hs_hbm.at[pl.ds(0, ECH)], rows_v.at[b],
                              ssems[b]).wait()

    for b in range(NBUF):
        prep_and_gather(b, b // 2, (b % 2) * ECH)

    def round_(o, carry):
        for b in range(NBUF):
            i = o * NBUF + b
            gwait(b)
            fire_scatter(b)
            br = (b - SDEP) % NBUF

            @pl.when(i >= SDEP)
            def _():
                swait(br)

            @pl.when(jnp.logical_and(i >= SDEP, i <= kch - 1 - (NBUF - SDEP)))
            def _():
                prep_and_gather(br, o * 2 + (b + SDEP) // 2, (b % 2) * ECH)
        return carry

    lax.fori_loop(0, kch // NBUF, round_, 0)
    for k in range(SDEP):
        swait((NBUF - SDEP + k) % NBUF)
    plsc.subcore_barrier()
    pltpu.sync_copy(acc.at[pl.ds(sid * RPT, RPT)],
                    out_hbm.at[cid, pl.ds(sid * RPT, RPT)])


_sc_agg = pl.kernel(
    _sc_agg_body,
    out_type=jax.ShapeDtypeStruct((NC, NPAD, C), jnp.float32),
    mesh=_mesh,
    scratch_types=[
        pltpu.VMEM((MAXK // 2, 2 * ECH), jnp.int32),
        pltpu.VMEM((NBUF, ECH), jnp.int32),
        pltpu.VMEM((NBUF, ECH), jnp.int32),
        pltpu.VMEM((NBUF, ECH, C), jnp.float32),
        pltpu.VMEM_SHARED((NPAD, C), jnp.float32),
        pltpu.SemaphoreType.DMA,
        pltpu.SemaphoreType.DMA,
        pltpu.SemaphoreType.DMA,
        pltpu.SemaphoreType.DMA,
        pltpu.SemaphoreType.DMA,
        pltpu.SemaphoreType.DMA,
        pltpu.SemaphoreType.DMA,
        pltpu.SemaphoreType.DMA,
    ],
)


BN = 2528


def _dinv_of(deg_ref):
    deg = deg_ref[0, :, 0:1] + deg_ref[1, :, 0:1] + 1.0
    return lax.rsqrt(deg)


def _tc1_body(x_ref, w_ref, deg_ref, hs_ref):
    h = jnp.dot(x_ref[...], w_ref[...], preferred_element_type=jnp.float32)
    hs_ref[...] = h * _dinv_of(deg_ref)


def _layernorm(t, g, b):
    mu = jnp.mean(t, axis=-1, keepdims=True)
    var = jnp.mean((t - mu) * (t - mu), axis=-1, keepdims=True)
    return (t - mu) * lax.rsqrt(var + 1e-5) * g + b


def _tc2_body(agg_ref, hs_ref, deg_ref, w1_ref, b0_ref, g0_ref, be0_ref,
              x1_ref, hs1_ref):
    dinv = _dinv_of(deg_ref)
    agg = agg_ref[0] + agg_ref[1]
    t = dinv * (agg + hs_ref[...]) + b0_ref[...]
    xb = jnp.maximum(_layernorm(t, g0_ref[...], be0_ref[...]), 0.0)
    x1_ref[...] = xb
    hs1_ref[...] = jnp.dot(xb, w1_ref[...],
                           preferred_element_type=jnp.float32) * dinv


BN3 = 2000


def _tc3_body(agg_ref, hs_ref, deg_ref, x1_ref, batch_ref,
              b1_ref, g1_ref, be1_ref, sums_ref, counts_ref):
    i = pl.program_id(0)

    @pl.when(i == 0)
    def _():
        sums_ref[...] = jnp.zeros_like(sums_ref)
        counts_ref[...] = jnp.zeros_like(counts_ref)

    dinv = _dinv_of(deg_ref)
    agg = agg_ref[0] + agg_ref[1]
    t = dinv * (agg + hs_ref[...]) + b1_ref[...]
    h2 = jnp.maximum(_layernorm(t, g1_ref[...], be1_ref[...]), 0.0) + x1_ref[...]

    iota = lax.broadcasted_iota(jnp.int32, (1, G), 1)
    oh = (batch_ref[...] == iota).astype(jnp.float32)
    dn = (((0,), (0,)), ((), ()))
    sums_ref[...] += lax.dot_general(oh, h2, dn,
                                     preferred_element_type=jnp.float32)
    counts_ref[...] += lax.dot_general(oh, jnp.ones((BN3, C), jnp.float32), dn,
                                       preferred_element_type=jnp.float32)


def _tc4_body(sums_ref, counts_ref, fw1_ref, fb1_ref, fw2_ref, fb2_ref,
              fw3_ref, fb3_ref, out_ref):
    pooled = sums_ref[...] / jnp.maximum(counts_ref[...], 1.0)
    z = jnp.maximum(jnp.dot(pooled, fw1_ref[...],
                            preferred_element_type=jnp.float32) + fb1_ref[...], 0.0)
    z = jnp.maximum(jnp.dot(z, fw2_ref[...],
                            preferred_element_type=jnp.float32) + fb2_ref[...], 0.0)
    z = jnp.dot(z, fw3_ref[...], preferred_element_type=jnp.float32) + fb3_ref[...]
    out_ref[...] = jnp.tanh(z)


def _row_spec(bn, ncols):
    return pl.BlockSpec((bn, ncols), lambda i: (i, 0))


def _half_spec(bn, ncols):
    return pl.BlockSpec((2, bn, ncols), lambda i: (0, i, 0))


def _full_spec(shape):
    return pl.BlockSpec(shape, lambda i: tuple(0 for _ in shape))


_tc1 = pl.pallas_call(
    _tc1_body,
    grid=(NPAD // BN,),
    in_specs=[_row_spec(BN, D), _full_spec((D, D)), _half_spec(BN, C)],
    out_specs=_row_spec(BN, D),
    out_shape=jax.ShapeDtypeStruct((NPAD, D), jnp.float32),
)

_tc2 = pl.pallas_call(
    _tc2_body,
    grid=(NPAD // BN,),
    in_specs=[_half_spec(BN, C), _row_spec(BN, C), _half_spec(BN, C),
              _full_spec((D, C)), _full_spec((1, D)), _full_spec((1, D)),
              _full_spec((1, D))],
    out_specs=[_row_spec(BN, D), _row_spec(BN, C)],
    out_shape=[jax.ShapeDtypeStruct((NPAD, D), jnp.float32),
               jax.ShapeDtypeStruct((NPAD, C), jnp.float32)],
)

_tc3 = pl.pallas_call(
    _tc3_body,
    grid=(N // BN3,),
    in_specs=[_half_spec(BN3, C), _row_spec(BN3, C), _half_spec(BN3, C),
              _row_spec(BN3, C), _row_spec(BN3, 1),
              _full_spec((1, C)), _full_spec((1, C)), _full_spec((1, C))],
    out_specs=[_full_spec((G, C)), _full_spec((G, C))],
    out_shape=[jax.ShapeDtypeStruct((G, C), jnp.float32),
               jax.ShapeDtypeStruct((G, C), jnp.float32)],
)

_tc4 = pl.pallas_call(
    _tc4_body,
    grid=(1,),
    in_specs=[_full_spec((G, C)), _full_spec((G, C)),
              _full_spec((C, FCW)), _full_spec((1, FCW)),
              _full_spec((FCW, FCW)), _full_spec((1, FCW)),
              _full_spec((FCW, C)), _full_spec((1, C))],
    out_specs=_full_spec((G, C)),
    out_shape=jax.ShapeDtypeStruct((G, C), jnp.float32),
)


@jax.jit
def kernel(x, edge_index, batch, W0, b0, g0, be0, W1, b1, g1, be1,
           fW1, fb1, fW2, fb2, fW3, fb3):
    src = edge_index[0].astype(jnp.int32)
    dst = edge_index[1].astype(jnp.int32)
    pad = jnp.full((EPAD - E,), N, jnp.int32)
    src_p = jnp.concatenate([src, pad])
    dst_p = jnp.concatenate([dst, pad])
    flat = src_p | (dst_p << 16)
    combo = flat.reshape(NW, NCH // 2, 2 * ECH)
    n0 = NS * K0 * ECH
    c0 = flat[:n0].reshape(NS, K0 * ECH)
    c1 = flat[n0:].reshape(NS, K1 * ECH)
    trash = jnp.int32(N | (N << 16))
    slab = jnp.full((NW, MAXK * ECH), trash, jnp.int32)
    slab = slab.at[0::2, :K0 * ECH].set(c0)
    slab = slab.at[1::2, :K1 * ECH].set(c1)
    combo_a = slab.reshape(NW, MAXK // 2, 2 * ECH)
    x_p = jnp.pad(x, ((0, NPAD - N), (0, 0)))

    zeros_r = jnp.zeros((RPT, C), jnp.float32)
    ones_c = jnp.ones((ECH, C), jnp.float32)

    deg2 = _sc_deg(combo, ones_c, zeros_r)

    hs0 = _tc1(x_p, W0, deg2)
    agg0 = _sc_agg(combo_a, hs0, zeros_r)
    x1, hs1 = _tc2(agg0, hs0, deg2, W1,
                   b0.reshape(1, D), g0.reshape(1, D), be0.reshape(1, D))
    agg1 = _sc_agg(combo_a, hs1, zeros_r)
    sums, counts = _tc3(agg1, hs1, deg2, x1, batch.reshape(N, 1).astype(jnp.int32),
                        b1.reshape(1, C), g1.reshape(1, C), be1.reshape(1, C))
    return _tc4(sums, counts, fW1, fb1.reshape(1, FCW), fW2, fb2.reshape(1, FCW),
                fW3, fb3.reshape(1, C))
